# baseline (speedup 1.0000x reference)
import jax
import jax.numpy as jnp
from jax import lax
from jax.experimental import pallas as pl
from jax.experimental.pallas import tpu as pltpu
from jax.experimental.pallas import tpu_sc as plsc

_NUM_FIELDS = 26
_VOCAB = 100000
_DIM = 64
_BATCH = 16384

_NC = 2
_NS = 16
_NW = _NC * _NS
_L = 16

_B_TOTAL = _BATCH * _NUM_FIELDS
_B_PER_W = _B_TOTAL // _NW
_CHUNK = 256
_N_CHUNKS = _B_PER_W // _CHUNK
_NBUF = 4


def _gather_body(idx_hbm, tab_hbm, out_hbm, idx_v, rows_v, *sems):
    gsem = sems[:_NBUF]
    wsem = sems[_NBUF:]
    wid = lax.axis_index("s") * _NC + lax.axis_index("c")
    base = wid * _B_PER_W

    pltpu.sync_copy(idx_hbm.at[pl.ds(base, _B_PER_W)], idx_v)

    def flatten_body(v, _):
        j0 = v * _L
        pos = j0 + lax.iota(jnp.int32, _L)
        field = lax.rem(pos, _NUM_FIELDS)
        idx = idx_v[pl.ds(j0, _L)]
        r = (
            field * (2 * _HPAD)
            + ((idx >> 7) << 7)
            + ((idx & 63) << 1)
            + ((idx >> 6) & 1)
        )
        idx_v[pl.ds(j0, _L)] = r
        return ()

    lax.fori_loop(0, _B_PER_W // _L, flatten_body, ())

    def fire_gather(c, b):
        pltpu.async_copy(
            tab_hbm.at[idx_v.at[pl.ds(c * _CHUNK, _CHUNK)]],
            rows_v.at[b],
            gsem[b],
        )

    def fire_write(c, b):
        pltpu.async_copy(
            rows_v.at[b],
            out_hbm.at[pl.ds(base + c * _CHUNK, _CHUNK)],
            wsem[b],
        )

    def wait_gather(c, b):
        pltpu.make_async_copy(
            tab_hbm.at[idx_v.at[pl.ds(c * _CHUNK, _CHUNK)]],
            rows_v.at[b],
            gsem[b],
        ).wait()

    def wait_write(c, b):
        pltpu.make_async_copy(
            rows_v.at[b],
            out_hbm.at[pl.ds(base + c * _CHUNK, _CHUNK)],
            wsem[b],
        ).wait()

    for b in range(_NBUF):
        fire_gather(b, b)

    @pl.loop(0, _N_CHUNKS - _NBUF, step=_NBUF)
    def _steady(c0):
        for b in range(_NBUF):
            c = c0 + b
            wait_gather(c, b)
            fire_write(c, b)
            wait_write(c, b)
            fire_gather(c + _NBUF, b)

    for b in range(_NBUF):
        c = _N_CHUNKS - _NBUF + b
        wait_gather(c, b)
        fire_write(c, b)
    for b in range(_NBUF):
        c = _N_CHUNKS - _NBUF + b
        wait_write(c, b)


_sc_gather = pl.kernel(
    _gather_body,
    out_type=jax.ShapeDtypeStruct((_B_TOTAL, _DIM), jnp.float32),
    mesh=plsc.VectorSubcoreMesh(core_axis_name="c", subcore_axis_name="s"),
    scratch_types=(
        [
            pltpu.VMEM((_B_PER_W,), jnp.int32),
            pltpu.VMEM((_NBUF, _CHUNK, _DIM), jnp.float32),
        ]
        + [pltpu.SemaphoreType.DMA] * (2 * _NBUF)
    ),
    compiler_params=pltpu.CompilerParams(use_tc_tiling_on_sc=False),
)


_VC = 1024
_N_VC = (_VOCAB + _VC - 1) // _VC
_VPAD = _N_VC * _VC
_HPAD = _VPAD // 2
_NSUB = _VC // 128


def _tp_body(a_ref, o_ref):
    a = a_ref[0]
    for t in range(_NSUB):
        lo = jnp.transpose(a[:, t * 128:t * 128 + 64], (1, 0))
        hi = jnp.transpose(a[:, t * 128 + 64:t * 128 + 128], (1, 0))
        o_ref[0, t * 64:t * 64 + 64, 0:_DIM] = lo
        o_ref[0, t * 64:t * 64 + 64, _DIM:2 * _DIM] = hi


_tc_transpose = pl.pallas_call(
    _tp_body,
    grid=(_NUM_FIELDS, _N_VC),
    in_specs=[pl.BlockSpec((1, _DIM, _VC), lambda f, c: (f, 0, c))],
    out_specs=pl.BlockSpec((1, _VC // 2, 2 * _DIM), lambda f, c: (f, c, 0)),
    out_shape=jax.ShapeDtypeStruct((_NUM_FIELDS, _HPAD, 2 * _DIM), jnp.float32),
)


def kernel(indexes, tables):
    idx_flat = indexes.reshape(_B_TOTAL).astype(jnp.int32)
    tab_t = jnp.transpose(tables, (0, 2, 1))
    tab_rm = _tc_transpose(tab_t)
    tab_flat = tab_rm.reshape(_NUM_FIELDS * 2 * _HPAD, _DIM)
    out = _sc_gather(idx_flat, tab_flat)
    return out.reshape(_BATCH, _NUM_FIELDS * _DIM)


# speedup vs baseline: 1.9050x; 1.9050x over previous
import jax
import jax.numpy as jnp
from jax import lax
from jax.experimental import pallas as pl
from jax.experimental.pallas import tpu as pltpu
from jax.experimental.pallas import tpu_sc as plsc

_NUM_FIELDS = 26
_VOCAB = 100000
_DIM = 64
_BATCH = 16384

_NC = 2
_NS = 16
_NW = _NC * _NS
_L = 16

_B_TOTAL = _BATCH * _NUM_FIELDS
_B_PER_W = _B_TOTAL // _NW
_CHUNK = 256
_N_CHUNKS = _B_PER_W // _CHUNK
_NBUF = 4


def _gather_body(idx_hbm, tab_hbm, out_hbm, idx_v, rows_v, *sems):
    gsem = sems[:_NBUF]
    wsem = sems[_NBUF:]
    wid = lax.axis_index("s") * _NC + lax.axis_index("c")
    base = wid * _B_PER_W

    pltpu.sync_copy(idx_hbm.at[pl.ds(base, _B_PER_W)], idx_v)

    def flatten_body(v, _):
        j0 = v * _L
        pos = j0 + lax.iota(jnp.int32, _L)
        field = lax.rem(pos, _NUM_FIELDS)
        idx = idx_v[pl.ds(j0, _L)]
        r = (
            field * (2 * _HPAD)
            + ((idx >> 7) << 7)
            + ((idx & 63) << 1)
            + ((idx >> 6) & 1)
        )
        idx_v[pl.ds(j0, _L)] = r
        return ()

    lax.fori_loop(0, _B_PER_W // _L, flatten_body, ())

    def fire_gather(c, b):
        pltpu.async_copy(
            tab_hbm.at[idx_v.at[pl.ds(c * _CHUNK, _CHUNK)]],
            rows_v.at[b],
            gsem[b],
        )

    def fire_write(c, b):
        pltpu.async_copy(
            rows_v.at[b],
            out_hbm.at[pl.ds(base + c * _CHUNK, _CHUNK)],
            wsem[b],
        )

    def wait_gather(c, b):
        pltpu.make_async_copy(
            tab_hbm.at[idx_v.at[pl.ds(c * _CHUNK, _CHUNK)]],
            rows_v.at[b],
            gsem[b],
        ).wait()

    def wait_write(c, b):
        pltpu.make_async_copy(
            rows_v.at[b],
            out_hbm.at[pl.ds(base + c * _CHUNK, _CHUNK)],
            wsem[b],
        ).wait()

    for b in range(_NBUF):
        fire_gather(b, b)

    @pl.loop(0, _N_CHUNKS - _NBUF, step=_NBUF)
    def _steady(c0):
        for b in range(_NBUF):
            c = c0 + b
            wait_gather(c, b)
            fire_write(c, b)
            wait_write(c, b)
            fire_gather(c + _NBUF, b)

    for b in range(_NBUF):
        c = _N_CHUNKS - _NBUF + b
        wait_gather(c, b)
        fire_write(c, b)
    for b in range(_NBUF):
        c = _N_CHUNKS - _NBUF + b
        wait_write(c, b)


_sc_gather = pl.kernel(
    _gather_body,
    out_type=jax.ShapeDtypeStruct((_B_TOTAL, _DIM), jnp.float32),
    mesh=plsc.VectorSubcoreMesh(core_axis_name="c", subcore_axis_name="s"),
    scratch_types=(
        [
            pltpu.VMEM((_B_PER_W,), jnp.int32),
            pltpu.VMEM((_NBUF, _CHUNK, _DIM), jnp.float32),
        ]
        + [pltpu.SemaphoreType.DMA] * (2 * _NBUF)
    ),
    compiler_params=pltpu.CompilerParams(use_tc_tiling_on_sc=False),
)


_VC = 2048
_N_VC = (_VOCAB + _VC - 1) // _VC
_VPAD = _N_VC * _VC
_HPAD = _VPAD // 2
_NSUB = _VC // 128


def _tp_body(a_ref, o_ref):
    a = a_ref[0]
    row = jax.lax.broadcasted_iota(jnp.int32, (_DIM, _DIM), 0)
    col = jax.lax.broadcasted_iota(jnp.int32, (_DIM, _DIM), 1)
    ident = (row == col).astype(jnp.float32)
    at = jax.lax.dot_general(
        a, ident, (((0,), (0,)), ((), ())),
        preferred_element_type=jnp.float32,
    )
    for t in range(_NSUB):
        o_ref[0, t * 64:t * 64 + 64, 0:_DIM] = at[t * 128:t * 128 + 64]
        o_ref[0, t * 64:t * 64 + 64, _DIM:2 * _DIM] = at[t * 128 + 64:t * 128 + 128]


_tc_transpose = pl.pallas_call(
    _tp_body,
    grid=(_NUM_FIELDS, _N_VC),
    in_specs=[pl.BlockSpec((1, _DIM, _VC), lambda f, c: (f, 0, c))],
    out_specs=pl.BlockSpec((1, _VC // 2, 2 * _DIM), lambda f, c: (f, c, 0)),
    out_shape=jax.ShapeDtypeStruct((_NUM_FIELDS, _HPAD, 2 * _DIM), jnp.float32),
)


def kernel(indexes, tables):
    idx_flat = indexes.reshape(_B_TOTAL).astype(jnp.int32)
    tab_t = jnp.transpose(tables, (0, 2, 1))
    tab_rm = _tc_transpose(tab_t)
    tab_flat = tab_rm.reshape(_NUM_FIELDS * 2 * _HPAD, _DIM)
    out = _sc_gather(idx_flat, tab_flat)
    return out.reshape(_BATCH, _NUM_FIELDS * _DIM)


# speedup vs baseline: 1.9741x; 1.0363x over previous
import jax
import jax.numpy as jnp
from jax import lax
from jax.experimental import pallas as pl
from jax.experimental.pallas import tpu as pltpu
from jax.experimental.pallas import tpu_sc as plsc

_NUM_FIELDS = 26
_VOCAB = 100000
_DIM = 64
_BATCH = 16384

_NC = 2
_NS = 16
_NW = _NC * _NS
_L = 16

_B_TOTAL = _BATCH * _NUM_FIELDS
_B_PER_W = _B_TOTAL // _NW
_CHUNK = 256
_N_CHUNKS = _B_PER_W // _CHUNK
_NBUF = 4


def _gather_body(idx_hbm, tab_hbm, out_hbm, idx_v, rows_v, *sems):
    gsem = sems[:_NBUF]
    wsem = sems[_NBUF:]
    wid = lax.axis_index("s") * _NC + lax.axis_index("c")
    base = wid * _B_PER_W

    pltpu.sync_copy(idx_hbm.at[pl.ds(base, _B_PER_W)], idx_v)

    def flatten_body(v, _):
        j0 = v * _L
        pos = j0 + lax.iota(jnp.int32, _L)
        field = lax.rem(pos, _NUM_FIELDS)
        idx = idx_v[pl.ds(j0, _L)]
        r = (
            field * (2 * _HPAD)
            + ((idx >> 11) << 11)
            + ((idx & 1023) << 1)
            + ((idx >> 10) & 1)
        )
        idx_v[pl.ds(j0, _L)] = r
        return ()

    lax.fori_loop(0, _B_PER_W // _L, flatten_body, ())

    def fire_gather(c, b):
        pltpu.async_copy(
            tab_hbm.at[idx_v.at[pl.ds(c * _CHUNK, _CHUNK)]],
            rows_v.at[b],
            gsem[b],
        )

    def fire_write(c, b):
        pltpu.async_copy(
            rows_v.at[b],
            out_hbm.at[pl.ds(base + c * _CHUNK, _CHUNK)],
            wsem[b],
        )

    def wait_gather(c, b):
        pltpu.make_async_copy(
            tab_hbm.at[idx_v.at[pl.ds(c * _CHUNK, _CHUNK)]],
            rows_v.at[b],
            gsem[b],
        ).wait()

    def wait_write(c, b):
        pltpu.make_async_copy(
            rows_v.at[b],
            out_hbm.at[pl.ds(base + c * _CHUNK, _CHUNK)],
            wsem[b],
        ).wait()

    for b in range(_NBUF):
        fire_gather(b, b)

    @pl.loop(0, _N_CHUNKS - _NBUF, step=_NBUF)
    def _steady(c0):
        for b in range(_NBUF):
            c = c0 + b
            wait_gather(c, b)
            fire_write(c, b)
            wait_write(c, b)
            fire_gather(c + _NBUF, b)

    for b in range(_NBUF):
        c = _N_CHUNKS - _NBUF + b
        wait_gather(c, b)
        fire_write(c, b)
    for b in range(_NBUF):
        c = _N_CHUNKS - _NBUF + b
        wait_write(c, b)


_sc_gather = pl.kernel(
    _gather_body,
    out_type=jax.ShapeDtypeStruct((_B_TOTAL, _DIM), jnp.float32),
    mesh=plsc.VectorSubcoreMesh(core_axis_name="c", subcore_axis_name="s"),
    scratch_types=(
        [
            pltpu.VMEM((_B_PER_W,), jnp.int32),
            pltpu.VMEM((_NBUF, _CHUNK, _DIM), jnp.float32),
        ]
        + [pltpu.SemaphoreType.DMA] * (2 * _NBUF)
    ),
    compiler_params=pltpu.CompilerParams(use_tc_tiling_on_sc=False),
)


_VC = 2048
_N_VC = (_VOCAB + _VC - 1) // _VC
_VPAD = _N_VC * _VC
_HPAD = _VPAD // 2


def _tp_body(a_ref, o_ref):
    a = a_ref[0]
    row = jax.lax.broadcasted_iota(jnp.int32, (_DIM, 2 * _DIM), 0)
    col = jax.lax.broadcasted_iota(jnp.int32, (_DIM, 2 * _DIM), 1)
    e_lo = (col == row).astype(jnp.float32)
    e_hi = (col == row + _DIM).astype(jnp.float32)
    dn = (((0,), (0,)), ((), ()))
    o_ref[0] = jax.lax.dot_general(
        a[:, : _VC // 2], e_lo, dn, preferred_element_type=jnp.float32
    ) + jax.lax.dot_general(
        a[:, _VC // 2 :], e_hi, dn, preferred_element_type=jnp.float32
    )


_tc_transpose = pl.pallas_call(
    _tp_body,
    grid=(_NUM_FIELDS, _N_VC),
    in_specs=[pl.BlockSpec((1, _DIM, _VC), lambda f, c: (f, 0, c))],
    out_specs=pl.BlockSpec((1, _VC // 2, 2 * _DIM), lambda f, c: (f, c, 0)),
    out_shape=jax.ShapeDtypeStruct((_NUM_FIELDS, _HPAD, 2 * _DIM), jnp.float32),
)


def kernel(indexes, tables):
    idx_flat = indexes.reshape(_B_TOTAL).astype(jnp.int32)
    tab_t = jnp.transpose(tables, (0, 2, 1))
    tab_rm = _tc_transpose(tab_t)
    tab_flat = tab_rm.reshape(_NUM_FIELDS * 2 * _HPAD, _DIM)
    out = _sc_gather(idx_flat, tab_flat)
    return out.reshape(_BATCH, _NUM_FIELDS * _DIM)


# speedup vs baseline: 2.6259x; 1.3302x over previous
import jax
import jax.numpy as jnp
from jax import lax
from jax.experimental import pallas as pl
from jax.experimental.pallas import tpu as pltpu
from jax.experimental.pallas import tpu_sc as plsc

_NUM_FIELDS = 26
_VOCAB = 100000
_DIM = 64
_BATCH = 16384

_NC = 2
_NS = 16
_NW = _NC * _NS
_L = 16

_B_TOTAL = _BATCH * _NUM_FIELDS
_B_PER_W = _B_TOTAL // _NW
_CHUNK = 256
_N_CHUNKS = _B_PER_W // _CHUNK
_NBUF = 4


def _gather_body(idx_hbm, tab_hbm, out_hbm, idx_v, rows_v, *sems):
    gsem = sems[:_NBUF]
    wsem = sems[_NBUF:]
    wid = lax.axis_index("s") * _NC + lax.axis_index("c")
    base = wid * _B_PER_W

    pltpu.sync_copy(idx_hbm.at[pl.ds(base, _B_PER_W)], idx_v)

    def flatten_body(v, _):
        j0 = v * _L
        pos = j0 + lax.iota(jnp.int32, _L)
        field = lax.rem(pos, _NUM_FIELDS)
        idx = idx_v[pl.ds(j0, _L)]
        r = (
            field * (2 * _HPAD)
            + ((idx >> 12) << 12)
            + ((idx & 2047) << 1)
            + ((idx >> 11) & 1)
        )
        idx_v[pl.ds(j0, _L)] = r
        return ()

    lax.fori_loop(0, _B_PER_W // _L, flatten_body, ())

    def fire_gather(c, b):
        pltpu.async_copy(
            tab_hbm.at[idx_v.at[pl.ds(c * _CHUNK, _CHUNK)]],
            rows_v.at[b],
            gsem[b],
        )

    def fire_write(c, b):
        pltpu.async_copy(
            rows_v.at[b],
            out_hbm.at[pl.ds(base + c * _CHUNK, _CHUNK)],
            wsem[b],
        )

    def wait_gather(c, b):
        pltpu.make_async_copy(
            tab_hbm.at[idx_v.at[pl.ds(c * _CHUNK, _CHUNK)]],
            rows_v.at[b],
            gsem[b],
        ).wait()

    def wait_write(c, b):
        pltpu.make_async_copy(
            rows_v.at[b],
            out_hbm.at[pl.ds(base + c * _CHUNK, _CHUNK)],
            wsem[b],
        ).wait()

    for b in range(_NBUF):
        fire_gather(b, b)

    @pl.loop(0, _N_CHUNKS - _NBUF, step=_NBUF)
    def _steady(c0):
        for b in range(_NBUF):
            c = c0 + b
            wait_gather(c, b)
            fire_write(c, b)
            wait_write(c, b)
            fire_gather(c + _NBUF, b)

    for b in range(_NBUF):
        c = _N_CHUNKS - _NBUF + b
        wait_gather(c, b)
        fire_write(c, b)
    for b in range(_NBUF):
        c = _N_CHUNKS - _NBUF + b
        wait_write(c, b)


_sc_gather = pl.kernel(
    _gather_body,
    out_type=jax.ShapeDtypeStruct((_B_TOTAL, _DIM), jnp.float32),
    mesh=plsc.VectorSubcoreMesh(core_axis_name="c", subcore_axis_name="s"),
    scratch_types=(
        [
            pltpu.VMEM((_B_PER_W,), jnp.int32),
            pltpu.VMEM((_NBUF, _CHUNK, _DIM), jnp.float32),
        ]
        + [pltpu.SemaphoreType.DMA] * (2 * _NBUF)
    ),
    compiler_params=pltpu.CompilerParams(use_tc_tiling_on_sc=False),
)


_VC = 4096
_N_VC = (_VOCAB + _VC - 1) // _VC
_VPAD = _N_VC * _VC
_HPAD = _VPAD // 2


def _tp_body(a_ref, o_ref):
    a = a_ref[0]
    row = jax.lax.broadcasted_iota(jnp.int32, (_DIM, 2 * _DIM), 0)
    col = jax.lax.broadcasted_iota(jnp.int32, (_DIM, 2 * _DIM), 1)
    e_lo = (col == row).astype(jnp.float32)
    e_hi = (col == row + _DIM).astype(jnp.float32)
    dn = (((0,), (0,)), ((), ()))
    o_ref[0] = jax.lax.dot_general(
        a[:, : _VC // 2], e_lo, dn, preferred_element_type=jnp.float32
    ) + jax.lax.dot_general(
        a[:, _VC // 2 :], e_hi, dn, preferred_element_type=jnp.float32
    )


_tc_transpose = pl.pallas_call(
    _tp_body,
    grid=(_NUM_FIELDS, _N_VC),
    in_specs=[pl.BlockSpec((1, _DIM, _VC), lambda f, c: (f, 0, c))],
    out_specs=pl.BlockSpec((1, _VC // 2, 2 * _DIM), lambda f, c: (f, c, 0)),
    out_shape=jax.ShapeDtypeStruct((_NUM_FIELDS, _HPAD, 2 * _DIM), jnp.float32),
)


def kernel(indexes, tables):
    idx_flat = indexes.reshape(_B_TOTAL).astype(jnp.int32)
    tab_t = jnp.transpose(tables, (0, 2, 1))
    tab_rm = _tc_transpose(tab_t)
    tab_flat = tab_rm.reshape(_NUM_FIELDS * 2 * _HPAD, _DIM)
    out = _sc_gather(idx_flat, tab_flat)
    return out.reshape(_BATCH, _NUM_FIELDS * _DIM)


# speedup vs baseline: 3.1852x; 1.2130x over previous
import jax
import jax.numpy as jnp
from jax import lax
from jax.experimental import pallas as pl
from jax.experimental.pallas import tpu as pltpu
from jax.experimental.pallas import tpu_sc as plsc

_NUM_FIELDS = 26
_VOCAB = 100000
_DIM = 64
_BATCH = 16384

_NC = 2
_NS = 16
_NW = _NC * _NS
_L = 16

_B_TOTAL = _BATCH * _NUM_FIELDS
_B_PER_W = _B_TOTAL // _NW
_CHUNK = 256
_N_CHUNKS = _B_PER_W // _CHUNK
_NBUF = 4


def _gather_body(idx_hbm, tab_hbm, out_hbm, idx_v, rows_v, *sems):
    gsem = sems[:_NBUF]
    wsem = sems[_NBUF:]
    wid = lax.axis_index("s") * _NC + lax.axis_index("c")
    base = wid * _B_PER_W

    pltpu.sync_copy(idx_hbm.at[pl.ds(base, _B_PER_W)], idx_v)

    def flatten_body(v, _):
        j0 = v * _L
        pos = j0 + lax.iota(jnp.int32, _L)
        field = lax.rem(pos, _NUM_FIELDS)
        idx = idx_v[pl.ds(j0, _L)]
        r = (
            field * (2 * _HPAD)
            + ((idx >> 13) << 13)
            + ((idx & 4095) << 1)
            + ((idx >> 12) & 1)
        )
        idx_v[pl.ds(j0, _L)] = r
        return ()

    lax.fori_loop(0, _B_PER_W // _L, flatten_body, ())

    def fire_gather(c, b):
        pltpu.async_copy(
            tab_hbm.at[idx_v.at[pl.ds(c * _CHUNK, _CHUNK)]],
            rows_v.at[b],
            gsem[b],
        )

    def fire_write(c, b):
        pltpu.async_copy(
            rows_v.at[b],
            out_hbm.at[pl.ds(base + c * _CHUNK, _CHUNK)],
            wsem[b],
        )

    def wait_gather(c, b):
        pltpu.make_async_copy(
            tab_hbm.at[idx_v.at[pl.ds(c * _CHUNK, _CHUNK)]],
            rows_v.at[b],
            gsem[b],
        ).wait()

    def wait_write(c, b):
        pltpu.make_async_copy(
            rows_v.at[b],
            out_hbm.at[pl.ds(base + c * _CHUNK, _CHUNK)],
            wsem[b],
        ).wait()

    for b in range(_NBUF):
        fire_gather(b, b)

    @pl.loop(0, _N_CHUNKS - _NBUF, step=_NBUF)
    def _steady(c0):
        for b in range(_NBUF):
            c = c0 + b
            wait_gather(c, b)
            fire_write(c, b)
            wait_write(c, b)
            fire_gather(c + _NBUF, b)

    for b in range(_NBUF):
        c = _N_CHUNKS - _NBUF + b
        wait_gather(c, b)
        fire_write(c, b)
    for b in range(_NBUF):
        c = _N_CHUNKS - _NBUF + b
        wait_write(c, b)


_sc_gather = pl.kernel(
    _gather_body,
    out_type=jax.ShapeDtypeStruct((_B_TOTAL, _DIM), jnp.float32),
    mesh=plsc.VectorSubcoreMesh(core_axis_name="c", subcore_axis_name="s"),
    scratch_types=(
        [
            pltpu.VMEM((_B_PER_W,), jnp.int32),
            pltpu.VMEM((_NBUF, _CHUNK, _DIM), jnp.float32),
        ]
        + [pltpu.SemaphoreType.DMA] * (2 * _NBUF)
    ),
    compiler_params=pltpu.CompilerParams(use_tc_tiling_on_sc=False),
)


_VC = 8192
_N_VC = (_VOCAB + _VC - 1) // _VC
_VPAD = _N_VC * _VC
_HPAD = _VPAD // 2


def _tp_body(a_ref, o_ref):
    a = a_ref[0]
    row = jax.lax.broadcasted_iota(jnp.int32, (_DIM, 2 * _DIM), 0)
    col = jax.lax.broadcasted_iota(jnp.int32, (_DIM, 2 * _DIM), 1)
    e_lo = (col == row).astype(jnp.float32)
    e_hi = (col == row + _DIM).astype(jnp.float32)
    dn = (((0,), (0,)), ((), ()))
    o_ref[0] = jax.lax.dot_general(
        a[:, : _VC // 2], e_lo, dn, preferred_element_type=jnp.float32
    ) + jax.lax.dot_general(
        a[:, _VC // 2 :], e_hi, dn, preferred_element_type=jnp.float32
    )


_tc_transpose = pl.pallas_call(
    _tp_body,
    grid=(_NUM_FIELDS, _N_VC),
    in_specs=[pl.BlockSpec((1, _DIM, _VC), lambda f, c: (f, 0, c))],
    out_specs=pl.BlockSpec((1, _VC // 2, 2 * _DIM), lambda f, c: (f, c, 0)),
    out_shape=jax.ShapeDtypeStruct((_NUM_FIELDS, _HPAD, 2 * _DIM), jnp.float32),
)


def kernel(indexes, tables):
    idx_flat = indexes.reshape(_B_TOTAL).astype(jnp.int32)
    tab_t = jnp.transpose(tables, (0, 2, 1))
    tab_rm = _tc_transpose(tab_t)
    tab_flat = tab_rm.reshape(_NUM_FIELDS * 2 * _HPAD, _DIM)
    out = _sc_gather(idx_flat, tab_flat)
    return out.reshape(_BATCH, _NUM_FIELDS * _DIM)


# speedup vs baseline: 3.4528x; 1.0840x over previous
import jax
import jax.numpy as jnp
from jax import lax
from jax.experimental import pallas as pl
from jax.experimental.pallas import tpu as pltpu
from jax.experimental.pallas import tpu_sc as plsc

_NUM_FIELDS = 26
_VOCAB = 100000
_DIM = 64
_BATCH = 16384

_NC = 2
_NS = 16
_NW = _NC * _NS
_L = 16

_B_TOTAL = _BATCH * _NUM_FIELDS
_B_PER_W = _B_TOTAL // _NW
_CHUNK = 256
_N_CHUNKS = _B_PER_W // _CHUNK
_NBUF = 4


def _gather_body(idx_hbm, tab_hbm, out_hbm, idx_v, rows_v, *sems):
    gsem = sems[:_NBUF]
    wsem = sems[_NBUF:]
    wid = lax.axis_index("s") * _NC + lax.axis_index("c")
    base = wid * _B_PER_W

    pltpu.sync_copy(idx_hbm.at[pl.ds(base, _B_PER_W)], idx_v)

    def flatten_body(v, _):
        j0 = v * _L
        pos = j0 + lax.iota(jnp.int32, _L)
        field = lax.rem(pos, _NUM_FIELDS)
        idx = idx_v[pl.ds(j0, _L)]
        r = (
            field * (2 * _HPAD)
            + ((idx >> 14) << 14)
            + ((idx & 8191) << 1)
            + ((idx >> 13) & 1)
        )
        idx_v[pl.ds(j0, _L)] = r
        return ()

    lax.fori_loop(0, _B_PER_W // _L, flatten_body, ())

    def fire_gather(c, b):
        pltpu.async_copy(
            tab_hbm.at[idx_v.at[pl.ds(c * _CHUNK, _CHUNK)]],
            rows_v.at[b],
            gsem[b],
        )

    def fire_write(c, b):
        pltpu.async_copy(
            rows_v.at[b],
            out_hbm.at[pl.ds(base + c * _CHUNK, _CHUNK)],
            wsem[b],
        )

    def wait_gather(c, b):
        pltpu.make_async_copy(
            tab_hbm.at[idx_v.at[pl.ds(c * _CHUNK, _CHUNK)]],
            rows_v.at[b],
            gsem[b],
        ).wait()

    def wait_write(c, b):
        pltpu.make_async_copy(
            rows_v.at[b],
            out_hbm.at[pl.ds(base + c * _CHUNK, _CHUNK)],
            wsem[b],
        ).wait()

    for b in range(_NBUF):
        fire_gather(b, b)

    @pl.loop(0, _N_CHUNKS - _NBUF, step=_NBUF)
    def _steady(c0):
        for b in range(_NBUF):
            c = c0 + b
            wait_gather(c, b)
            fire_write(c, b)
            wait_write(c, b)
            fire_gather(c + _NBUF, b)

    for b in range(_NBUF):
        c = _N_CHUNKS - _NBUF + b
        wait_gather(c, b)
        fire_write(c, b)
    for b in range(_NBUF):
        c = _N_CHUNKS - _NBUF + b
        wait_write(c, b)


_sc_gather = pl.kernel(
    _gather_body,
    out_type=jax.ShapeDtypeStruct((_B_TOTAL, _DIM), jnp.float32),
    mesh=plsc.VectorSubcoreMesh(core_axis_name="c", subcore_axis_name="s"),
    scratch_types=(
        [
            pltpu.VMEM((_B_PER_W,), jnp.int32),
            pltpu.VMEM((_NBUF, _CHUNK, _DIM), jnp.float32),
        ]
        + [pltpu.SemaphoreType.DMA] * (2 * _NBUF)
    ),
    compiler_params=pltpu.CompilerParams(use_tc_tiling_on_sc=False),
)


_VC = 16384
_N_VC = (_VOCAB + _VC - 1) // _VC
_VPAD = _N_VC * _VC
_HPAD = _VPAD // 2


def _tp_body(a_ref, o_ref):
    a = a_ref[0]
    row = jax.lax.broadcasted_iota(jnp.int32, (_DIM, 2 * _DIM), 0)
    col = jax.lax.broadcasted_iota(jnp.int32, (_DIM, 2 * _DIM), 1)
    e_lo = (col == row).astype(jnp.float32)
    e_hi = (col == row + _DIM).astype(jnp.float32)
    dn = (((0,), (0,)), ((), ()))
    o_ref[0] = jax.lax.dot_general(
        a[:, : _VC // 2], e_lo, dn, preferred_element_type=jnp.float32
    ) + jax.lax.dot_general(
        a[:, _VC // 2 :], e_hi, dn, preferred_element_type=jnp.float32
    )


_tc_transpose = pl.pallas_call(
    _tp_body,
    grid=(_NUM_FIELDS, _N_VC),
    in_specs=[pl.BlockSpec((1, _DIM, _VC), lambda f, c: (f, 0, c))],
    out_specs=pl.BlockSpec((1, _VC // 2, 2 * _DIM), lambda f, c: (f, c, 0)),
    out_shape=jax.ShapeDtypeStruct((_NUM_FIELDS, _HPAD, 2 * _DIM), jnp.float32),
)


def kernel(indexes, tables):
    idx_flat = indexes.reshape(_B_TOTAL).astype(jnp.int32)
    tab_t = jnp.transpose(tables, (0, 2, 1))
    tab_rm = _tc_transpose(tab_t)
    tab_flat = tab_rm.reshape(_NUM_FIELDS * 2 * _HPAD, _DIM)
    out = _sc_gather(idx_flat, tab_flat)
    return out.reshape(_BATCH, _NUM_FIELDS * _DIM)
